# Initial kernel scaffold; baseline (speedup 1.0000x reference)
import jax
import jax.numpy as jnp
from jax.experimental import pallas as pl


def kernel(InState, GnnPerms, NNsites, gdiags, Psi0, bias0, Psi1, bias1, Psi2, bias2, Psi3, bias3, Psi4, bias4, wtVC):
    raise NotImplementedError("write your pallas kernel here")


# speedup vs baseline: 3.1107x; 3.1107x over previous
import jax
import jax.numpy as jnp
from jax.experimental import pallas as pl
from jax.experimental.pallas import tpu as pltpu

_NG = 48
_NNGB = 13
_S = 1024
_NB = 128
_DIM = 3
_NBP = 8
_CHANS = [(2, 8), (8, 8), (8, 8), (8, 8), (8, 1)]

_f32 = jnp.float32
_bf16 = jnp.bfloat16


def _split(x):
    hi = x.astype(_bf16)
    lo = (x - hi.astype(_f32)).astype(_bf16)
    return hi, lo


def _softplus(y):
    return jnp.maximum(y, 0.0) + jnp.log1p(jnp.exp(-jnp.abs(y)))


def _dot3(wh, wl, rh, rl):
    return (
        jnp.dot(wh, rh, preferred_element_type=_f32)
        + jnp.dot(wh, rl, preferred_element_type=_f32)
        + jnp.dot(wl, rh, preferred_element_type=_f32)
    )


def _body(x0_ref, a_ref, gdh_ref, gdl_ref, wph_ref, wpl_ref, *rest):
    nw = 2 * len(_CHANS)
    w_refs = rest[:nw]
    b_refs = rest[nw : nw + len(_CHANS)]
    out_ref = rest[nw + len(_CHANS)]
    rh_ref, rl_ref, x_ref = rest[nw + len(_CHANS) + 1 :]

    A = a_ref[...]
    x_ref[: _CHANS[0][0] * _NBP, :] = x0_ref[...].reshape(
        _CHANS[0][0] * _NBP, _S
    )

    for l, (C, O) in enumerate(_CHANS):
        CN = C * _NBP
        hi, lo = _split(x_ref[:CN, :])
        stack = jnp.concatenate([hi, lo], axis=0)
        G = jnp.dot(stack, A, preferred_element_type=_f32)
        for j in range(_NNGB):
            blk = G[:, j * _S : (j + 1) * _S].astype(_bf16)
            rh_ref[:, j * C : (j + 1) * C, :] = blk[:CN].reshape(_NBP, C, _S)
            rl_ref[:, j * C : (j + 1) * C, :] = blk[CN:].reshape(_NBP, C, _S)
        Wh = w_refs[2 * l][...]
        Wl = w_refs[2 * l + 1][...]
        Gb = b_refs[l][...]

        def conv_b(b, _, C=C, O=O, Wh=Wh, Wl=Wl, Gb=Gb):
            rh = rh_ref[b, : _NNGB * C, :]
            rl = rl_ref[b, : _NNGB * C, :]
            Y = _dot3(Wh, Wl, rh, rl) + Gb
            Sp = _softplus(Y).reshape(O, _NG, _S)
            x_ref[pl.ds(b * O, O), :] = jnp.sum(Sp, axis=1) * (1.0 / _NG)
            return _

        jax.lax.fori_loop(0, _NBP, conv_b, 0)

    hi, lo = _split(x_ref[:_NBP, :])
    stack = jnp.concatenate([hi, lo], axis=0)
    G = jnp.dot(stack, A, preferred_element_type=_f32)
    for j in range(_NNGB):
        blk = G[:, j * _S : (j + 1) * _S].astype(_bf16)
        rh_ref[:, j, :] = blk[:_NBP]
        rl_ref[:, j, :] = blk[_NBP:]

    T = _dot3(gdh_ref[...], gdl_ref[...], wph_ref[...], wpl_ref[...])
    Th, Tl = _split(T)

    def out_b(b, _):
        rh = rh_ref[b, :_NNGB, :]
        rl = rl_ref[b, :_NNGB, :]
        Yb = _dot3(Th, Tl, rh, rl)
        out_ref[b] = jnp.sum(Yb.reshape(_NG, _DIM, _S), axis=0) * (1.0 / _NG)
        return _

    jax.lax.fori_loop(0, _NBP, out_b, 0)


def kernel(InState, GnnPerms, NNsites, gdiags, Psi0, bias0, Psi1, bias1,
           Psi2, bias2, Psi3, bias3, Psi4, bias4, wtVC):
    Psis = [Psi0, Psi1, Psi2, Psi3, Psi4]
    biases = [bias0, bias1, bias2, bias3, bias4]

    nnflat = NNsites.reshape(_NNGB * _S)
    A = (nnflat[None, :] == jnp.arange(_S, dtype=nnflat.dtype)[:, None]).astype(_bf16)

    w_list, b_list = [], []
    for (C, O), Psi, bias in zip(_CHANS, Psis, biases):
        wrep = jnp.repeat(Psi, _NG, axis=0)
        perm = jnp.tile(GnnPerms, (O, C)).reshape(-1, C, _NNGB)
        GW = jnp.take_along_axis(wrep, perm, axis=2)
        GWp = GW.transpose(0, 2, 1).reshape(O * _NG, _NNGB * C)
        wh, wl = _split(GWp)
        w_list += [wh, wl]
        b_list.append(jnp.repeat(bias, _NG, axis=0))

    wt_rep = jnp.tile(wtVC, (_NG, 1))
    perm = jnp.repeat(GnnPerms, _DIM, axis=0)
    wtp = jnp.take_along_axis(wt_rep, perm, axis=1)
    wph, wpl = _split(wtp)
    gdh, gdl = _split(gdiags)

    grid = (_NB // _NBP,)

    in_specs = [
        pl.BlockSpec((_NBP, _CHANS[0][0], _S), lambda i: (i, 0, 0)),
        pl.BlockSpec((_S, _NNGB * _S), lambda i: (0, 0)),
        pl.BlockSpec((_NG * _DIM, _NG * _DIM), lambda i: (0, 0)),
        pl.BlockSpec((_NG * _DIM, _NG * _DIM), lambda i: (0, 0)),
        pl.BlockSpec((_NG * _DIM, _NNGB), lambda i: (0, 0)),
        pl.BlockSpec((_NG * _DIM, _NNGB), lambda i: (0, 0)),
    ]
    ops = [InState, A, gdh, gdl, wph, wpl]
    for w in w_list:
        in_specs.append(pl.BlockSpec(w.shape, lambda i: (0, 0)))
        ops.append(w)
    for bb in b_list:
        in_specs.append(pl.BlockSpec(bb.shape, lambda i: (0, 0)))
        ops.append(bb)

    out = pl.pallas_call(
        _body,
        grid=grid,
        in_specs=in_specs,
        out_specs=pl.BlockSpec((_NBP, _DIM, _S), lambda i: (i, 0, 0)),
        out_shape=jax.ShapeDtypeStruct((_NB, _DIM, _S), _f32),
        scratch_shapes=[
            pltpu.VMEM((_NBP, _NNGB * 8, _S), _bf16),
            pltpu.VMEM((_NBP, _NNGB * 8, _S), _bf16),
            pltpu.VMEM((_NBP * 8, _S), _f32),
        ],
        compiler_params=pltpu.CompilerParams(
            dimension_semantics=("arbitrary",),
            vmem_limit_bytes=64 * 1024 * 1024,
        ),
    )(*ops)
    return out


# speedup vs baseline: 3.1211x; 1.0033x over previous
import jax
import jax.numpy as jnp
from jax.experimental import pallas as pl
from jax.experimental.pallas import tpu as pltpu

_NG = 48
_NNGB = 13
_S = 1024
_NB = 128
_DIM = 3
_NBP = 16
_CHANS = [(2, 8), (8, 8), (8, 8), (8, 8), (8, 1)]

_f32 = jnp.float32
_bf16 = jnp.bfloat16


def _split(x):
    hi = x.astype(_bf16)
    lo = (x - hi.astype(_f32)).astype(_bf16)
    return hi, lo


def _softplus(y):
    return jnp.maximum(y, 0.0) + jnp.log1p(jnp.exp(-jnp.abs(y)))


def _dot3(wh, wl, rh, rl):
    return (
        jnp.dot(wh, rh, preferred_element_type=_f32)
        + jnp.dot(wh, rl, preferred_element_type=_f32)
        + jnp.dot(wl, rh, preferred_element_type=_f32)
    )


def _body(x0_ref, nn_ref, gdh_ref, gdl_ref, wph_ref, wpl_ref, *rest):
    nw = 2 * len(_CHANS)
    w_refs = rest[:nw]
    b_refs = rest[nw : nw + len(_CHANS)]
    out_ref = rest[nw + len(_CHANS)]
    a_ref, rh_ref, rl_ref, x_ref = rest[nw + len(_CHANS) + 1 :]

    @pl.when(pl.program_id(0) == 0)
    def _build_a():
        nn = jnp.broadcast_to(nn_ref[...], (_S, _NNGB * _S))
        row = jax.lax.broadcasted_iota(jnp.int32, (_S, _NNGB * _S), 0)
        a_ref[...] = (nn == row).astype(_bf16)

    x_ref[: _CHANS[0][0] * _NBP, :] = x0_ref[...].reshape(
        _CHANS[0][0] * _NBP, _S
    )

    for l, (C, O) in enumerate(_CHANS):
        CN = C * _NBP
        hi, lo = _split(x_ref[:CN, :])
        stack = jnp.concatenate([hi, lo], axis=0)
        for j in range(_NNGB):
            Aj = a_ref[:, j * _S : (j + 1) * _S]
            Gj = jnp.dot(stack, Aj, preferred_element_type=_f32).astype(_bf16)
            rh_ref[:, j * C : (j + 1) * C, :] = Gj[:CN].reshape(_NBP, C, _S)
            rl_ref[:, j * C : (j + 1) * C, :] = Gj[CN:].reshape(_NBP, C, _S)
        Wh = w_refs[2 * l][...]
        Wl = w_refs[2 * l + 1][...]
        Gb = b_refs[l][...]

        def conv_b(b, _, C=C, O=O, Wh=Wh, Wl=Wl, Gb=Gb):
            rh = rh_ref[b, : _NNGB * C, :]
            rl = rl_ref[b, : _NNGB * C, :]
            Y = _dot3(Wh, Wl, rh, rl) + Gb
            Sp = _softplus(Y).reshape(O, _NG, _S)
            x_ref[pl.ds(b * O, O), :] = jnp.sum(Sp, axis=1) * (1.0 / _NG)
            return _

        jax.lax.fori_loop(0, _NBP, conv_b, 0)

    hi, lo = _split(x_ref[:_NBP, :])
    stack = jnp.concatenate([hi, lo], axis=0)
    for j in range(_NNGB):
        Aj = a_ref[:, j * _S : (j + 1) * _S]
        Gj = jnp.dot(stack, Aj, preferred_element_type=_f32).astype(_bf16)
        rh_ref[:, j, :] = Gj[:_NBP]
        rl_ref[:, j, :] = Gj[_NBP:]

    T = _dot3(gdh_ref[...], gdl_ref[...], wph_ref[...], wpl_ref[...])
    Th, Tl = _split(T)

    def out_b(b, _):
        rh = rh_ref[b, :_NNGB, :]
        rl = rl_ref[b, :_NNGB, :]
        Yb = _dot3(Th, Tl, rh, rl)
        out_ref[b] = jnp.sum(Yb.reshape(_NG, _DIM, _S), axis=0) * (1.0 / _NG)
        return _

    jax.lax.fori_loop(0, _NBP, out_b, 0)


def kernel(InState, GnnPerms, NNsites, gdiags, Psi0, bias0, Psi1, bias1,
           Psi2, bias2, Psi3, bias3, Psi4, bias4, wtVC):
    Psis = [Psi0, Psi1, Psi2, Psi3, Psi4]
    biases = [bias0, bias1, bias2, bias3, bias4]

    nnflat = NNsites.astype(jnp.int32).reshape(1, _NNGB * _S)

    w_list, b_list = [], []
    for (C, O), Psi, bias in zip(_CHANS, Psis, biases):
        wrep = jnp.repeat(Psi, _NG, axis=0)
        perm = jnp.tile(GnnPerms, (O, C)).reshape(-1, C, _NNGB)
        GW = jnp.take_along_axis(wrep, perm, axis=2)
        GWp = GW.transpose(0, 2, 1).reshape(O * _NG, _NNGB * C)
        wh, wl = _split(GWp)
        w_list += [wh, wl]
        b_list.append(jnp.repeat(bias, _NG, axis=0))

    wt_rep = jnp.tile(wtVC, (_NG, 1))
    perm = jnp.repeat(GnnPerms, _DIM, axis=0)
    wtp = jnp.take_along_axis(wt_rep, perm, axis=1)
    wph, wpl = _split(wtp)
    gdh, gdl = _split(gdiags)

    grid = (_NB // _NBP,)

    in_specs = [
        pl.BlockSpec((_NBP, _CHANS[0][0], _S), lambda i: (i, 0, 0)),
        pl.BlockSpec((1, _NNGB * _S), lambda i: (0, 0)),
        pl.BlockSpec((_NG * _DIM, _NG * _DIM), lambda i: (0, 0)),
        pl.BlockSpec((_NG * _DIM, _NG * _DIM), lambda i: (0, 0)),
        pl.BlockSpec((_NG * _DIM, _NNGB), lambda i: (0, 0)),
        pl.BlockSpec((_NG * _DIM, _NNGB), lambda i: (0, 0)),
    ]
    ops = [InState, nnflat, gdh, gdl, wph, wpl]
    for w in w_list:
        in_specs.append(pl.BlockSpec(w.shape, lambda i: (0, 0)))
        ops.append(w)
    for bb in b_list:
        in_specs.append(pl.BlockSpec(bb.shape, lambda i: (0, 0)))
        ops.append(bb)

    out = pl.pallas_call(
        _body,
        grid=grid,
        in_specs=in_specs,
        out_specs=pl.BlockSpec((_NBP, _DIM, _S), lambda i: (i, 0, 0)),
        out_shape=jax.ShapeDtypeStruct((_NB, _DIM, _S), _f32),
        scratch_shapes=[
            pltpu.VMEM((_S, _NNGB * _S), _bf16),
            pltpu.VMEM((_NBP, _NNGB * 8, _S), _bf16),
            pltpu.VMEM((_NBP, _NNGB * 8, _S), _bf16),
            pltpu.VMEM((_NBP * 8, _S), _f32),
        ],
        compiler_params=pltpu.CompilerParams(
            dimension_semantics=("arbitrary",),
            vmem_limit_bytes=64 * 1024 * 1024,
        ),
    )(*ops)
    return out
